# baseline (speedup 1.0000x reference)
import functools

import jax
import jax.numpy as jnp
from jax import lax
from jax.experimental import pallas as pl
from jax.experimental.pallas import tpu as pltpu
from jax.experimental.pallas import tpu_sc as plsc

B, C, N, K = 8, 256, 16384, 512
NB = 2048
L = 16
NUM_CORES, NUM_SUBCORES = 2, 16
TPB = 4
CPT = K // TPB
NCH = N // L
CC = 32


def _dist2_body(h_ref, pie_ref, out_ref):
    hb = h_ref[0]
    d = hb - pie_ref[0]
    s = jnp.sum(d * d, axis=0)
    out_ref[0, 0, :] = lax.bitcast_convert_type(s, jnp.int32)


def _dist2(h, pie):
    return pl.pallas_call(
        _dist2_body,
        grid=(N // NB, B),
        in_specs=[
            pl.BlockSpec((1, C, NB), lambda j, b: (b, 0, j)),
            pl.BlockSpec((1, C, 1), lambda j, b: (b, 0, 0)),
        ],
        out_specs=pl.BlockSpec((1, 1, NB), lambda j, b: (b, 0, j)),
        out_shape=jax.ShapeDtypeStruct((B, 1, N), jnp.int32),
    )(h, pie)


def _sc_body(dist2_hbm, h8_hbm, out_hbm,
             d_v, hist_v, bufa_v, bufb_v, sel_v, list_v, dst_v, cmp_v, sem):
    cid = lax.axis_index("c")
    sid = lax.axis_index("s")
    wid = cid * NUM_SUBCORES + sid
    b = wid // TPB
    q = wid % TPB

    pltpu.sync_copy(dist2_hbm.at[b], d_v)

    iota = lax.iota(jnp.int32, L)
    ones = jnp.ones((L,), jnp.int32)
    zeros = jnp.zeros((L,), jnp.int32)

    def read_d(i):
        return d_v[pl.ds(i * L, L)]

    def level(read, cnt, shift, remaining, dst):
        for i in range(16):
            hist_v[pl.ds(i * L, L)] = zeros
        nch = (cnt + L - 1) // L

        def hbody(i, carry):
            v = read(i)
            act = (i * L + iota) < cnt
            digit = lax.shift_right_logical(v, shift) & 255
            plsc.addupdate_scatter(hist_v, [digit], ones, mask=act)
            return carry

        lax.fori_loop(0, nch, hbody, jnp.int32(0))

        def sbody(j, carry):
            base, beta, below = carry
            cm = base + plsc.cumsum(hist_v[pl.ds(j * L, L)])
            lt = cm < remaining
            beta = beta + jnp.max(plsc.all_reduce_population_count(lt))
            below = jnp.maximum(below, jnp.max(jnp.where(lt, cm, zeros)))
            return jnp.max(cm), beta, below

        _, beta, below = lax.fori_loop(
            0, 16, sbody, (jnp.int32(0), jnp.int32(0), jnp.int32(0)))
        remaining = remaining - below

        if dst is None:
            return beta, remaining, cnt

        def fbody(i, off):
            v = read(i)
            act = ((i * L + iota) < cnt) & (
                (lax.shift_right_logical(v, shift) & 255) == beta)
            plsc.store_compressed(dst.at[pl.ds(off, L)], v, mask=act)
            return off + jnp.max(plsc.all_reduce_population_count(act))

        newcnt = lax.fori_loop(0, nch, fbody, jnp.int32(0))
        return beta, remaining, newcnt

    read_a = lambda i: bufa_v[pl.ds(i * L, L)]
    read_b = lambda i: bufb_v[pl.ds(i * L, L)]

    beta0, rem, cnt1 = level(read_d, N, 24, jnp.int32(K), bufa_v)
    beta1, rem, cnt2 = level(read_a, cnt1, 16, rem, bufb_v)
    beta2, rem, cnt3 = level(read_b, cnt2, 8, rem, bufa_v)
    beta3, rem, _ = level(read_a, cnt3, 0, rem, None)
    vbits = ((beta0 * 256 + beta1) * 256 + beta2) * 256 + beta3

    def ebody(i, carry):
        offl, offt = carry
        v = read_d(i)
        lane = i * L + iota
        actl = v < vbits
        plsc.store_compressed(sel_v.at[pl.ds(offl, L)], lane, mask=actl)
        offl = offl + jnp.max(plsc.all_reduce_population_count(actl))
        actt = v == vbits
        plsc.store_compressed(bufb_v.at[pl.ds(offt, L)], lane, mask=actt)
        offt = offt + jnp.max(plsc.all_reduce_population_count(actt))
        return offl, offt

    n_less, _ = lax.fori_loop(0, NCH, ebody, (jnp.int32(0), jnp.int32(0)))

    def cbody(j, carry):
        sel_v[pl.ds(n_less + j * L, L)] = bufb_v[pl.ds(j * L, L)]
        return carry

    lax.fori_loop(0, (rem + L - 1) // L, cbody, jnp.int32(0))

    base_b = b * (C * N)

    def chunk_body(g, carry):
        c0 = g * CC

        def lbody(c, carry):
            base8 = (base_b + (c0 + c) * N) // 8
            for j in range(CPT // L):
                kv = sel_v[pl.ds(q * CPT + j * L, L)]
                list_v[c, pl.ds(j * L, L)] = (
                    base8 + lax.shift_right_logical(kv, 3))
            return carry

        lax.fori_loop(0, CC, lbody, jnp.int32(0))

        def gstart(c, carry):
            pltpu.async_copy(h8_hbm.at[list_v.at[c]], dst_v.at[c], sem)
            return carry

        lax.fori_loop(0, CC, gstart, jnp.int32(0))

        def gwait(c, carry):
            pltpu.make_async_copy(
                h8_hbm.at[list_v.at[c]], dst_v.at[c], sem).wait()
            return carry

        lax.fori_loop(0, CC, gwait, jnp.int32(0))

        def ebody(c, carry):
            cvec = jnp.broadcast_to(c, (L,))
            for j in range(CPT // L):
                kv = sel_v[pl.ds(q * CPT + j * L, L)]
                v = plsc.load_gather(
                    dst_v, [cvec, j * L + iota, kv & 7])
                cmp_v[c, pl.ds(j * L, L)] = v
            return carry

        lax.fori_loop(0, CC, ebody, jnp.int32(0))

        pltpu.sync_copy(cmp_v, out_hbm.at[b, g, q])
        return carry

    lax.fori_loop(0, C // CC, chunk_body, jnp.int32(0))


@functools.lru_cache(maxsize=1)
def _sc_topk_gather():
    mesh = plsc.VectorSubcoreMesh(
        core_axis_name="c", subcore_axis_name="s",
        num_cores=NUM_CORES, num_subcores=NUM_SUBCORES)
    return pl.kernel(
        _sc_body,
        out_type=jax.ShapeDtypeStruct((B, C // CC, TPB, CC, CPT),
                                      jnp.float32),
        mesh=mesh,
        compiler_params=pltpu.CompilerParams(
            needs_layout_passes=False, use_tc_tiling_on_sc=False),
        scratch_types=[
            pltpu.VMEM((N,), jnp.int32),
            pltpu.VMEM((256,), jnp.int32),
            pltpu.VMEM((N + L,), jnp.int32),
            pltpu.VMEM((N + L,), jnp.int32),
            pltpu.VMEM((K + 2 * L,), jnp.int32),
            pltpu.VMEM((CC, CPT), jnp.int32),
            pltpu.VMEM((CC, CPT, 8), jnp.float32),
            pltpu.VMEM((CC, CPT), jnp.float32),
            pltpu.SemaphoreType.DMA,
        ],
    )


def _attn_body(nb_ref, pie_ref, wc_ref, bc_ref, wa_ref, ba_ref, out_ref):
    rel = nb_ref[0] - pie_ref[0]
    t = jnp.dot(wc_ref[...], rel,
                preferred_element_type=jnp.float32) + bc_ref[...][:, None]
    s = lax.dot_general(t, t, (((0,), (0,)), ((), ())),
                        preferred_element_type=jnp.float32)
    e = jnp.exp(s - jnp.max(s, axis=1, keepdims=True))
    z = jnp.sum(e, axis=1, keepdims=True)
    w = (jnp.sum(e / z, axis=0) * (1.0 / K))[:, None]
    feat = jnp.dot(t, w, preferred_element_type=jnp.float32)
    o = jnp.dot(wa_ref[...], feat,
                preferred_element_type=jnp.float32)[:, 0] + ba_ref[...]
    out_ref[0, 0, :] = jnp.maximum(o, 0.0)


def _attn(nbrs, pie, w_conv, b_conv, w_att, b_att):
    return pl.pallas_call(
        _attn_body,
        grid=(B,),
        in_specs=[
            pl.BlockSpec((1, C, K), lambda b: (b, 0, 0)),
            pl.BlockSpec((1, C, 1), lambda b: (b, 0, 0)),
            pl.BlockSpec((C, C), lambda b: (0, 0)),
            pl.BlockSpec((C,), lambda b: (0,)),
            pl.BlockSpec((C, C), lambda b: (0, 0)),
            pl.BlockSpec((C,), lambda b: (0,)),
        ],
        out_specs=pl.BlockSpec((1, 1, C), lambda b: (b, 0, 0)),
        out_shape=jax.ShapeDtypeStruct((B, 1, C), jnp.float32),
    )(nbrs, pie, w_conv, b_conv, w_att, b_att)


def kernel(h, pi, W_conv, b_conv, W_att, b_att):
    pie = pi[:, :, None]
    dist2 = _dist2(h, pie).reshape(B, N)
    h8 = h.reshape(B * C * N // 8, 8)
    raw = _sc_topk_gather()(dist2, h8)
    nbrs = raw.reshape(B, C // CC, TPB, CC, CPT).transpose(
        0, 1, 3, 2, 4).reshape(B, C, K)
    return _attn(nbrs, pie, W_conv, b_conv, W_att, b_att).reshape(B, C)


# speedup vs baseline: 1.0161x; 1.0161x over previous
import functools

import jax
import jax.numpy as jnp
from jax import lax
from jax.experimental import pallas as pl
from jax.experimental.pallas import tpu as pltpu
from jax.experimental.pallas import tpu_sc as plsc

B, C, N, K = 8, 256, 16384, 512
NB = 2048
L = 16
NUM_CORES, NUM_SUBCORES = 2, 16
TPB = 4
CPT = K // TPB
NCH = N // L
CC = 32


def _dist2_body(h_ref, pie_ref, out_ref):
    hb = h_ref[0]
    d = hb - pie_ref[0]
    s = jnp.sum(d * d, axis=0)
    out_ref[0, 0, :] = lax.bitcast_convert_type(s, jnp.int32)


def _dist2(h, pie):
    return pl.pallas_call(
        _dist2_body,
        grid=(N // NB, B),
        in_specs=[
            pl.BlockSpec((1, C, NB), lambda j, b: (b, 0, j)),
            pl.BlockSpec((1, C, 1), lambda j, b: (b, 0, 0)),
        ],
        out_specs=pl.BlockSpec((1, 1, NB), lambda j, b: (b, 0, j)),
        out_shape=jax.ShapeDtypeStruct((B, 1, N), jnp.int32),
    )(h, pie)


def _sc_body(dist2_hbm, h8_hbm, out_hbm,
             d_v, hist_v, bufa_v, bufb_v, sel_v, list_v, dst_v, cmp_v, sem):
    cid = lax.axis_index("c")
    sid = lax.axis_index("s")
    wid = cid * NUM_SUBCORES + sid
    b = wid // TPB
    q = wid % TPB

    pltpu.sync_copy(dist2_hbm.at[b], d_v)

    iota = lax.iota(jnp.int32, L)
    ones = jnp.ones((L,), jnp.int32)
    zeros = jnp.zeros((L,), jnp.int32)

    def read_d(i):
        return d_v[pl.ds(i * L, L)]

    def level(read, cnt, shift, remaining, dst):
        for i in range(16):
            hist_v[pl.ds(i * L, L)] = zeros
        nch = (cnt + L - 1) // L

        def hbody(i, carry):
            v = read(i)
            act = (i * L + iota) < cnt
            digit = lax.shift_right_logical(v, shift) & 255
            plsc.addupdate_scatter(hist_v, [digit], ones, mask=act)
            return carry

        lax.fori_loop(0, nch, hbody, jnp.int32(0))

        def sbody(j, carry):
            base, beta, below = carry
            cm = base + plsc.cumsum(hist_v[pl.ds(j * L, L)])
            lt = cm < remaining
            beta = beta + jnp.max(plsc.all_reduce_population_count(lt))
            below = jnp.maximum(below, jnp.max(jnp.where(lt, cm, zeros)))
            return jnp.max(cm), beta, below

        _, beta, below = lax.fori_loop(
            0, 16, sbody, (jnp.int32(0), jnp.int32(0), jnp.int32(0)))
        remaining = remaining - below

        if dst is None:
            return beta, remaining, cnt

        def fbody(i, off):
            v = read(i)
            act = ((i * L + iota) < cnt) & (
                (lax.shift_right_logical(v, shift) & 255) == beta)
            plsc.store_compressed(dst.at[pl.ds(off, L)], v, mask=act)
            return off + jnp.max(plsc.all_reduce_population_count(act))

        newcnt = lax.fori_loop(0, nch, fbody, jnp.int32(0))
        return beta, remaining, newcnt

    read_a = lambda i: bufa_v[pl.ds(i * L, L)]
    read_b = lambda i: bufb_v[pl.ds(i * L, L)]

    beta0, rem, cnt1 = level(read_d, N, 24, jnp.int32(K), bufa_v)
    beta1, rem, cnt2 = level(read_a, cnt1, 16, rem, bufb_v)
    beta2, rem, cnt3 = level(read_b, cnt2, 8, rem, bufa_v)
    beta3, rem, _ = level(read_a, cnt3, 0, rem, None)
    vbits = ((beta0 * 256 + beta1) * 256 + beta2) * 256 + beta3

    def ebody(i, carry):
        offl, offt = carry
        v = read_d(i)
        lane = i * L + iota
        actl = v < vbits
        plsc.store_compressed(sel_v.at[pl.ds(offl, L)], lane, mask=actl)
        offl = offl + jnp.max(plsc.all_reduce_population_count(actl))
        actt = v == vbits
        plsc.store_compressed(bufb_v.at[pl.ds(offt, L)], lane, mask=actt)
        offt = offt + jnp.max(plsc.all_reduce_population_count(actt))
        return offl, offt

    n_less, _ = lax.fori_loop(0, NCH, ebody, (jnp.int32(0), jnp.int32(0)))

    def cbody(j, carry):
        sel_v[pl.ds(n_less + j * L, L)] = bufb_v[pl.ds(j * L, L)]
        return carry

    lax.fori_loop(0, (rem + L - 1) // L, cbody, jnp.int32(0))

    base_b = b * (C * N)

    def chunk_body(g, carry):
        c0 = g * CC

        def lbody(c, carry):
            base8 = (base_b + (c0 + c) * N) // 8
            for j in range(CPT // L):
                kv = sel_v[pl.ds(q * CPT + j * L, L)]
                list_v[c, pl.ds(j * L, L)] = (
                    base8 + lax.shift_right_logical(kv, 3))
            return carry

        lax.fori_loop(0, CC, lbody, jnp.int32(0))

        def gstart(c, carry):
            pltpu.async_copy(h8_hbm.at[list_v.at[c]], dst_v.at[c], sem)
            return carry

        lax.fori_loop(0, CC, gstart, jnp.int32(0))

        def gwait(c, carry):
            pltpu.make_async_copy(
                h8_hbm.at[list_v.at[c]], dst_v.at[c], sem).wait()
            return carry

        lax.fori_loop(0, CC, gwait, jnp.int32(0))

        def ebody(c, carry):
            cvec = jnp.broadcast_to(c, (L,))
            for j in range(CPT // L):
                kv = sel_v[pl.ds(q * CPT + j * L, L)]
                v = plsc.load_gather(
                    dst_v, [cvec, j * L + iota, kv & 7])
                cmp_v[c, pl.ds(j * L, L)] = v
            return carry

        lax.fori_loop(0, CC, ebody, jnp.int32(0))

        pltpu.sync_copy(
            cmp_v, out_hbm.at[b, pl.ds(c0, CC), pl.ds(q * CPT, CPT)])
        return carry

    lax.fori_loop(0, C // CC, chunk_body, jnp.int32(0))


@functools.lru_cache(maxsize=1)
def _sc_topk_gather():
    mesh = plsc.VectorSubcoreMesh(
        core_axis_name="c", subcore_axis_name="s",
        num_cores=NUM_CORES, num_subcores=NUM_SUBCORES)
    return pl.kernel(
        _sc_body,
        out_type=jax.ShapeDtypeStruct((B, C, K), jnp.float32),
        mesh=mesh,
        compiler_params=pltpu.CompilerParams(
            needs_layout_passes=False, use_tc_tiling_on_sc=False),
        scratch_types=[
            pltpu.VMEM((N,), jnp.int32),
            pltpu.VMEM((256,), jnp.int32),
            pltpu.VMEM((N + L,), jnp.int32),
            pltpu.VMEM((N + L,), jnp.int32),
            pltpu.VMEM((K + 2 * L,), jnp.int32),
            pltpu.VMEM((CC, CPT), jnp.int32),
            pltpu.VMEM((CC, CPT, 8), jnp.float32),
            pltpu.VMEM((CC, CPT), jnp.float32),
            pltpu.SemaphoreType.DMA,
        ],
    )


def _attn_body(nb_ref, pie_ref, wc_ref, bc_ref, wa_ref, ba_ref, out_ref):
    rel = nb_ref[0] - pie_ref[0]
    t = jnp.dot(wc_ref[...], rel,
                preferred_element_type=jnp.float32) + bc_ref[...][:, None]
    s = lax.dot_general(t, t, (((0,), (0,)), ((), ())),
                        preferred_element_type=jnp.float32)
    e = jnp.exp(s - jnp.max(s, axis=1, keepdims=True))
    z = jnp.sum(e, axis=1, keepdims=True)
    w = (jnp.sum(e / z, axis=0) * (1.0 / K))[:, None]
    feat = jnp.dot(t, w, preferred_element_type=jnp.float32)
    o = jnp.dot(wa_ref[...], feat,
                preferred_element_type=jnp.float32)[:, 0] + ba_ref[...]
    out_ref[0, 0, :] = jnp.maximum(o, 0.0)


def _attn(nbrs, pie, w_conv, b_conv, w_att, b_att):
    return pl.pallas_call(
        _attn_body,
        grid=(B,),
        in_specs=[
            pl.BlockSpec((1, C, K), lambda b: (b, 0, 0)),
            pl.BlockSpec((1, C, 1), lambda b: (b, 0, 0)),
            pl.BlockSpec((C, C), lambda b: (0, 0)),
            pl.BlockSpec((C,), lambda b: (0,)),
            pl.BlockSpec((C, C), lambda b: (0, 0)),
            pl.BlockSpec((C,), lambda b: (0,)),
        ],
        out_specs=pl.BlockSpec((1, 1, C), lambda b: (b, 0, 0)),
        out_shape=jax.ShapeDtypeStruct((B, 1, C), jnp.float32),
    )(nbrs, pie, w_conv, b_conv, w_att, b_att)


def kernel(h, pi, W_conv, b_conv, W_att, b_att):
    pie = pi[:, :, None]
    dist2 = _dist2(h, pie).reshape(B, N)
    h8 = h.reshape(B * C * N // 8, 8)
    nbrs = _sc_topk_gather()(dist2, h8)
    return _attn(nbrs, pie, W_conv, b_conv, W_att, b_att).reshape(B, C)


# speedup vs baseline: 1.1321x; 1.1141x over previous
import functools

import jax
import jax.numpy as jnp
from jax import lax
from jax.experimental import pallas as pl
from jax.experimental.pallas import tpu as pltpu
from jax.experimental.pallas import tpu_sc as plsc

B, C, N, K = 8, 256, 16384, 512
NB = 2048
L = 16
NUM_CORES, NUM_SUBCORES = 2, 16
TPB = 4
CPT = K // TPB
NCH = N // L
CC = 32


def _dist2_body(h_ref, pie_ref, out_ref):
    hb = h_ref[0]
    d = hb - pie_ref[0]
    s = jnp.sum(d * d, axis=0)
    out_ref[0, 0, :] = lax.bitcast_convert_type(s, jnp.int32)


def _dist2(h, pie):
    return pl.pallas_call(
        _dist2_body,
        grid=(N // NB, B),
        in_specs=[
            pl.BlockSpec((1, C, NB), lambda j, b: (b, 0, j)),
            pl.BlockSpec((1, C, 1), lambda j, b: (b, 0, 0)),
        ],
        out_specs=pl.BlockSpec((1, 1, NB), lambda j, b: (b, 0, j)),
        out_shape=jax.ShapeDtypeStruct((B, 1, N), jnp.int32),
    )(h, pie)


def _sc_body(dist2_hbm, h8_hbm, out_hbm, d_v, hist_v, bufa_v, bufb_v,
             sel_v, list_v, dst_v, cmp_v, sem_a, sem_b):
    cid = lax.axis_index("c")
    sid = lax.axis_index("s")
    wid = cid * NUM_SUBCORES + sid
    b = wid // TPB
    q = wid % TPB

    pltpu.sync_copy(dist2_hbm.at[b], d_v)

    iota = lax.iota(jnp.int32, L)
    ones = jnp.ones((L,), jnp.int32)
    zeros = jnp.zeros((L,), jnp.int32)

    def read_d(i):
        return d_v[pl.ds(i * L, L)]

    def level(read, cnt, shift, remaining, dst):
        for i in range(16):
            hist_v[pl.ds(i * L, L)] = zeros
        nch = (cnt + L - 1) // L

        def hbody(i, carry):
            v = read(i)
            act = (i * L + iota) < cnt
            digit = lax.shift_right_logical(v, shift) & 255
            plsc.addupdate_scatter(hist_v, [digit], ones, mask=act)
            return carry

        lax.fori_loop(0, nch, hbody, jnp.int32(0))

        def sbody(j, carry):
            base, beta, below = carry
            cm = base + plsc.cumsum(hist_v[pl.ds(j * L, L)])
            lt = cm < remaining
            beta = beta + jnp.max(plsc.all_reduce_population_count(lt))
            below = jnp.maximum(below, jnp.max(jnp.where(lt, cm, zeros)))
            return jnp.max(cm), beta, below

        _, beta, below = lax.fori_loop(
            0, 16, sbody, (jnp.int32(0), jnp.int32(0), jnp.int32(0)))
        remaining = remaining - below

        if dst is None:
            return beta, remaining, cnt

        def fbody(i, off):
            v = read(i)
            act = ((i * L + iota) < cnt) & (
                (lax.shift_right_logical(v, shift) & 255) == beta)
            plsc.store_compressed(dst.at[pl.ds(off, L)], v, mask=act)
            return off + jnp.max(plsc.all_reduce_population_count(act))

        newcnt = lax.fori_loop(0, nch, fbody, jnp.int32(0))
        return beta, remaining, newcnt

    read_a = lambda i: bufa_v[pl.ds(i * L, L)]
    read_b = lambda i: bufb_v[pl.ds(i * L, L)]

    beta0, rem, cnt1 = level(read_d, N, 24, jnp.int32(K), bufa_v)
    beta1, rem, cnt2 = level(read_a, cnt1, 16, rem, bufb_v)
    beta2, rem, cnt3 = level(read_b, cnt2, 8, rem, bufa_v)
    beta3, rem, _ = level(read_a, cnt3, 0, rem, None)
    vbits = ((beta0 * 256 + beta1) * 256 + beta2) * 256 + beta3

    def ebody(i, carry):
        offl, offt = carry
        v = read_d(i)
        lane = i * L + iota
        actl = v < vbits
        plsc.store_compressed(sel_v.at[pl.ds(offl, L)], lane, mask=actl)
        offl = offl + jnp.max(plsc.all_reduce_population_count(actl))
        actt = v == vbits
        plsc.store_compressed(bufb_v.at[pl.ds(offt, L)], lane, mask=actt)
        offt = offt + jnp.max(plsc.all_reduce_population_count(actt))
        return offl, offt

    n_less, _ = lax.fori_loop(0, NCH, ebody, (jnp.int32(0), jnp.int32(0)))

    def cbody(j, carry):
        sel_v[pl.ds(n_less + j * L, L)] = bufb_v[pl.ds(j * L, L)]
        return carry

    lax.fori_loop(0, (rem + L - 1) // L, cbody, jnp.int32(0))

    base_b = b * (C * N)
    sems = (sem_a, sem_b)
    G = C // CC

    def fill(g, p):
        def lbody(c, carry):
            base8 = (base_b + (g * CC + c) * N) // 8
            for j in range(CPT // L):
                kv = sel_v[pl.ds(q * CPT + j * L, L)]
                list_v[p, c, pl.ds(j * L, L)] = (
                    base8 + lax.shift_right_logical(kv, 3))
            return carry

        lax.fori_loop(0, CC, lbody, jnp.int32(0))

        def gstart(c, carry):
            pltpu.async_copy(
                h8_hbm.at[list_v.at[p, c]], dst_v.at[p, c], sems[p])
            return carry

        lax.fori_loop(0, CC, gstart, jnp.int32(0))

    def drain(g, p):
        def gwait(c, carry):
            pltpu.make_async_copy(
                h8_hbm.at[list_v.at[p, c]], dst_v.at[p, c], sems[p]).wait()
            return carry

        lax.fori_loop(0, CC, gwait, jnp.int32(0))

        def ebody(c, carry):
            pvec = jnp.broadcast_to(jnp.int32(p), (L,))
            cvec = jnp.broadcast_to(c, (L,))
            for j in range(CPT // L):
                kv = sel_v[pl.ds(q * CPT + j * L, L)]
                v = plsc.load_gather(
                    dst_v, [pvec, cvec, j * L + iota, kv & 7])
                cmp_v[c, pl.ds(j * L, L)] = v
            return carry

        lax.fori_loop(0, CC, ebody, jnp.int32(0))

        pltpu.sync_copy(
            cmp_v, out_hbm.at[b, pl.ds(g * CC, CC), pl.ds(q * CPT, CPT)])

    fill(jnp.int32(0), 0)

    def pair_body(i, carry):
        g = 2 * i
        fill(g + 1, 1)
        drain(g, 0)
        fill(g + 2, 0)
        drain(g + 1, 1)
        return carry

    lax.fori_loop(0, G // 2 - 1, pair_body, jnp.int32(0))
    fill(jnp.int32(G - 1), 1)
    drain(jnp.int32(G - 2), 0)
    drain(jnp.int32(G - 1), 1)


@functools.lru_cache(maxsize=1)
def _sc_topk_gather():
    mesh = plsc.VectorSubcoreMesh(
        core_axis_name="c", subcore_axis_name="s",
        num_cores=NUM_CORES, num_subcores=NUM_SUBCORES)
    return pl.kernel(
        _sc_body,
        out_type=jax.ShapeDtypeStruct((B, C, K), jnp.float32),
        mesh=mesh,
        compiler_params=pltpu.CompilerParams(
            needs_layout_passes=False, use_tc_tiling_on_sc=False),
        scratch_types=[
            pltpu.VMEM((N,), jnp.int32),
            pltpu.VMEM((256,), jnp.int32),
            pltpu.VMEM((N + L,), jnp.int32),
            pltpu.VMEM((N + L,), jnp.int32),
            pltpu.VMEM((K + 2 * L,), jnp.int32),
            pltpu.VMEM((2, CC, CPT), jnp.int32),
            pltpu.VMEM((2, CC, CPT, 8), jnp.float32),
            pltpu.VMEM((CC, CPT), jnp.float32),
            pltpu.SemaphoreType.DMA,
            pltpu.SemaphoreType.DMA,
        ],
    )


def _attn_body(nb_ref, pie_ref, wc_ref, bc_ref, wa_ref, ba_ref, out_ref):
    rel = nb_ref[0] - pie_ref[0]
    t = jnp.dot(wc_ref[...], rel,
                preferred_element_type=jnp.float32) + bc_ref[...][:, None]
    s = lax.dot_general(t, t, (((0,), (0,)), ((), ())),
                        preferred_element_type=jnp.float32)
    e = jnp.exp(s - jnp.max(s, axis=1, keepdims=True))
    z = jnp.sum(e, axis=1, keepdims=True)
    w = (jnp.sum(e / z, axis=0) * (1.0 / K))[:, None]
    feat = jnp.dot(t, w, preferred_element_type=jnp.float32)
    o = jnp.dot(wa_ref[...], feat,
                preferred_element_type=jnp.float32)[:, 0] + ba_ref[...]
    out_ref[0, 0, :] = jnp.maximum(o, 0.0)


def _attn(nbrs, pie, w_conv, b_conv, w_att, b_att):
    return pl.pallas_call(
        _attn_body,
        grid=(B,),
        in_specs=[
            pl.BlockSpec((1, C, K), lambda b: (b, 0, 0)),
            pl.BlockSpec((1, C, 1), lambda b: (b, 0, 0)),
            pl.BlockSpec((C, C), lambda b: (0, 0)),
            pl.BlockSpec((C,), lambda b: (0,)),
            pl.BlockSpec((C, C), lambda b: (0, 0)),
            pl.BlockSpec((C,), lambda b: (0,)),
        ],
        out_specs=pl.BlockSpec((1, 1, C), lambda b: (b, 0, 0)),
        out_shape=jax.ShapeDtypeStruct((B, 1, C), jnp.float32),
    )(nbrs, pie, w_conv, b_conv, w_att, b_att)


def kernel(h, pi, W_conv, b_conv, W_att, b_att):
    pie = pi[:, :, None]
    dist2 = _dist2(h, pie).reshape(B, N)
    h8 = h.reshape(B * C * N // 8, 8)
    nbrs = _sc_topk_gather()(dist2, h8)
    return _attn(nbrs, pie, W_conv, b_conv, W_att, b_att).reshape(B, C)


# speedup vs baseline: 1.2564x; 1.1098x over previous
import functools

import jax
import jax.numpy as jnp
from jax import lax
from jax.experimental import pallas as pl
from jax.experimental.pallas import tpu as pltpu
from jax.experimental.pallas import tpu_sc as plsc

B, C, N, K = 8, 256, 16384, 512
NB = 2048
L = 16
NUM_CORES, NUM_SUBCORES = 2, 16
TPB = 4
CPT = K // TPB
NCH = N // L
CC = 32


def _dist2_body(h_ref, pie_ref, out_ref, hlin_ref):
    hb = h_ref[0]
    d = hb - pie_ref[0]
    s = jnp.sum(d * d, axis=0)
    out_ref[0, 0, :] = lax.bitcast_convert_type(s, jnp.int32)
    hlin_ref[0] = hb.reshape(C, NB // 128, 128)


def _dist2(h, pie):
    return pl.pallas_call(
        _dist2_body,
        grid=(N // NB, B),
        in_specs=[
            pl.BlockSpec((1, C, NB), lambda j, b: (b, 0, j)),
            pl.BlockSpec((1, C, 1), lambda j, b: (b, 0, 0)),
        ],
        out_specs=[
            pl.BlockSpec((1, 1, NB), lambda j, b: (b, 0, j)),
            pl.BlockSpec((1, C, NB // 128, 128), lambda j, b: (b, 0, j, 0)),
        ],
        out_shape=[
            jax.ShapeDtypeStruct((B, 1, N), jnp.int32),
            jax.ShapeDtypeStruct((B, C, N // 128, 128), jnp.float32),
        ],
    )(h, pie)


def _sc_body(dist2_hbm, h8_hbm, out_hbm, d_v, hist_v, bufa_v, bufb_v,
             sel_v, list_v, dst_v, cmp_v, sem_a, sem_b):
    cid = lax.axis_index("c")
    sid = lax.axis_index("s")
    wid = cid * NUM_SUBCORES + sid
    b = wid // TPB
    q = wid % TPB

    pltpu.sync_copy(dist2_hbm.at[b], d_v)

    iota = lax.iota(jnp.int32, L)
    ones = jnp.ones((L,), jnp.int32)
    zeros = jnp.zeros((L,), jnp.int32)

    def read_d(i):
        return d_v[pl.ds(i * L, L)]

    def level(read, cnt, shift, remaining, dst):
        for i in range(16):
            hist_v[pl.ds(i * L, L)] = zeros
        nch = (cnt + L - 1) // L

        def hbody(i, carry):
            v = read(i)
            act = (i * L + iota) < cnt
            digit = lax.shift_right_logical(v, shift) & 255
            plsc.addupdate_scatter(hist_v, [digit], ones, mask=act)
            return carry

        lax.fori_loop(0, nch, hbody, jnp.int32(0))

        def sbody(j, carry):
            base, beta, below = carry
            cm = base + plsc.cumsum(hist_v[pl.ds(j * L, L)])
            lt = cm < remaining
            beta = beta + jnp.max(plsc.all_reduce_population_count(lt))
            below = jnp.maximum(below, jnp.max(jnp.where(lt, cm, zeros)))
            return jnp.max(cm), beta, below

        _, beta, below = lax.fori_loop(
            0, 16, sbody, (jnp.int32(0), jnp.int32(0), jnp.int32(0)))
        remaining = remaining - below

        if dst is None:
            return beta, remaining, cnt

        def fbody(i, off):
            v = read(i)
            act = ((i * L + iota) < cnt) & (
                (lax.shift_right_logical(v, shift) & 255) == beta)
            plsc.store_compressed(dst.at[pl.ds(off, L)], v, mask=act)
            return off + jnp.max(plsc.all_reduce_population_count(act))

        newcnt = lax.fori_loop(0, nch, fbody, jnp.int32(0))
        return beta, remaining, newcnt

    read_a = lambda i: bufa_v[pl.ds(i * L, L)]
    read_b = lambda i: bufb_v[pl.ds(i * L, L)]

    beta0, rem, cnt1 = level(read_d, N, 24, jnp.int32(K), bufa_v)
    beta1, rem, cnt2 = level(read_a, cnt1, 16, rem, bufb_v)
    beta2, rem, cnt3 = level(read_b, cnt2, 8, rem, bufa_v)
    beta3, rem, _ = level(read_a, cnt3, 0, rem, None)
    vbits = ((beta0 * 256 + beta1) * 256 + beta2) * 256 + beta3

    def ebody(i, carry):
        offl, offt = carry
        v = read_d(i)
        lane = i * L + iota
        actl = v < vbits
        plsc.store_compressed(sel_v.at[pl.ds(offl, L)], lane, mask=actl)
        offl = offl + jnp.max(plsc.all_reduce_population_count(actl))
        actt = v == vbits
        plsc.store_compressed(bufb_v.at[pl.ds(offt, L)], lane, mask=actt)
        offt = offt + jnp.max(plsc.all_reduce_population_count(actt))
        return offl, offt

    n_less, _ = lax.fori_loop(0, NCH, ebody, (jnp.int32(0), jnp.int32(0)))

    def cbody(j, carry):
        sel_v[pl.ds(n_less + j * L, L)] = bufb_v[pl.ds(j * L, L)]
        return carry

    lax.fori_loop(0, (rem + L - 1) // L, cbody, jnp.int32(0))

    base_b = b * (C * N)
    sems = (sem_a, sem_b)
    G = C // CC

    def fill(g, p):
        def lbody(c, carry):
            base8 = (base_b + (g * CC + c) * N) // 8
            for j in range(CPT // L):
                kv = sel_v[pl.ds(q * CPT + j * L, L)]
                list_v[p, c, pl.ds(j * L, L)] = (
                    base8 + lax.shift_right_logical(kv, 3))
            return carry

        lax.fori_loop(0, CC, lbody, jnp.int32(0))

        def gstart(c, carry):
            pltpu.async_copy(
                h8_hbm.at[list_v.at[p, c]], dst_v.at[p, c], sems[p])
            return carry

        lax.fori_loop(0, CC, gstart, jnp.int32(0))

    def drain(g, p):
        def gwait(c, carry):
            pltpu.make_async_copy(
                h8_hbm.at[list_v.at[p, c]], dst_v.at[p, c], sems[p]).wait()
            return carry

        lax.fori_loop(0, CC, gwait, jnp.int32(0))

        def ebody(c, carry):
            pvec = jnp.broadcast_to(jnp.int32(p), (L,))
            cvec = jnp.broadcast_to(c, (L,))
            for j in range(CPT // L):
                kv = sel_v[pl.ds(q * CPT + j * L, L)]
                v = plsc.load_gather(
                    dst_v, [pvec, cvec, j * L + iota, kv & 7])
                cmp_v[c, pl.ds(j * L, L)] = v
            return carry

        lax.fori_loop(0, CC, ebody, jnp.int32(0))

        pltpu.sync_copy(
            cmp_v, out_hbm.at[b, pl.ds(g * CC, CC), pl.ds(q * CPT, CPT)])

    fill(jnp.int32(0), 0)

    def pair_body(i, carry):
        g = 2 * i
        fill(g + 1, 1)
        drain(g, 0)
        fill(g + 2, 0)
        drain(g + 1, 1)
        return carry

    lax.fori_loop(0, G // 2 - 1, pair_body, jnp.int32(0))
    fill(jnp.int32(G - 1), 1)
    drain(jnp.int32(G - 2), 0)
    drain(jnp.int32(G - 1), 1)


@functools.lru_cache(maxsize=1)
def _sc_topk_gather():
    mesh = plsc.VectorSubcoreMesh(
        core_axis_name="c", subcore_axis_name="s",
        num_cores=NUM_CORES, num_subcores=NUM_SUBCORES)
    return pl.kernel(
        _sc_body,
        out_type=jax.ShapeDtypeStruct((B, C, K), jnp.float32),
        mesh=mesh,
        compiler_params=pltpu.CompilerParams(
            needs_layout_passes=False, use_tc_tiling_on_sc=False),
        scratch_types=[
            pltpu.VMEM((N,), jnp.int32),
            pltpu.VMEM((256,), jnp.int32),
            pltpu.VMEM((N + L,), jnp.int32),
            pltpu.VMEM((N + L,), jnp.int32),
            pltpu.VMEM((K + 2 * L,), jnp.int32),
            pltpu.VMEM((2, CC, CPT), jnp.int32),
            pltpu.VMEM((2, CC, CPT, 8), jnp.float32),
            pltpu.VMEM((CC, CPT), jnp.float32),
            pltpu.SemaphoreType.DMA,
            pltpu.SemaphoreType.DMA,
        ],
    )


def _attn_body(nb_ref, pie_ref, wc_ref, bc_ref, wa_ref, ba_ref, out_ref):
    rel = nb_ref[0] - pie_ref[0]
    t = jnp.dot(wc_ref[...], rel,
                preferred_element_type=jnp.float32) + bc_ref[...][:, None]
    s = lax.dot_general(t, t, (((0,), (0,)), ((), ())),
                        preferred_element_type=jnp.float32)
    e = jnp.exp(s - jnp.max(s, axis=1, keepdims=True))
    z = jnp.sum(e, axis=1, keepdims=True)
    w = (jnp.sum(e / z, axis=0) * (1.0 / K))[:, None]
    feat = jnp.dot(t, w, preferred_element_type=jnp.float32)
    o = jnp.dot(wa_ref[...], feat,
                preferred_element_type=jnp.float32)[:, 0] + ba_ref[...]
    out_ref[0, 0, :] = jnp.maximum(o, 0.0)


def _attn(nbrs, pie, w_conv, b_conv, w_att, b_att):
    return pl.pallas_call(
        _attn_body,
        grid=(B,),
        in_specs=[
            pl.BlockSpec((1, C, K), lambda b: (b, 0, 0)),
            pl.BlockSpec((1, C, 1), lambda b: (b, 0, 0)),
            pl.BlockSpec((C, C), lambda b: (0, 0)),
            pl.BlockSpec((C,), lambda b: (0,)),
            pl.BlockSpec((C, C), lambda b: (0, 0)),
            pl.BlockSpec((C,), lambda b: (0,)),
        ],
        out_specs=pl.BlockSpec((1, 1, C), lambda b: (b, 0, 0)),
        out_shape=jax.ShapeDtypeStruct((B, 1, C), jnp.float32),
    )(nbrs, pie, w_conv, b_conv, w_att, b_att)


def kernel(h, pi, W_conv, b_conv, W_att, b_att):
    pie = pi[:, :, None]
    dist2_r, hlin = _dist2(h, pie)
    dist2 = dist2_r.reshape(B, N)
    h8 = hlin.reshape(B * C * N // 8, 8)
    nbrs = _sc_topk_gather()(dist2, h8)
    return _attn(nbrs, pie, W_conv, b_conv, W_att, b_att).reshape(B, C)


# speedup vs baseline: 1.5817x; 1.2589x over previous
import functools

import jax
import jax.numpy as jnp
from jax import lax
from jax.experimental import pallas as pl
from jax.experimental.pallas import tpu as pltpu
from jax.experimental.pallas import tpu_sc as plsc

B, C, N, K = 8, 256, 16384, 512
NB = 2048
L = 16
NUM_CORES, NUM_SUBCORES = 2, 16
TPB = 4
CPT = K // TPB
NCH = N // L
CC = 32


def _dist2_body(h_ref, pie_ref, out_ref, ht_ref):
    hb = h_ref[0]
    d = hb - pie_ref[0]
    s = jnp.sum(d * d, axis=0)
    out_ref[0, 0, :] = lax.bitcast_convert_type(s, jnp.int32)
    ht_ref[0] = jnp.transpose(hb).reshape(NB * 2, 128)


def _dist2(h, pie):
    return pl.pallas_call(
        _dist2_body,
        grid=(N // NB, B),
        in_specs=[
            pl.BlockSpec((1, C, NB), lambda j, b: (b, 0, j)),
            pl.BlockSpec((1, C, 1), lambda j, b: (b, 0, 0)),
        ],
        out_specs=[
            pl.BlockSpec((1, 1, NB), lambda j, b: (b, 0, j)),
            pl.BlockSpec((1, NB * 2, 128), lambda j, b: (b, j, 0)),
        ],
        out_shape=[
            jax.ShapeDtypeStruct((B, 1, N), jnp.int32),
            jax.ShapeDtypeStruct((B, N * 2, 128), jnp.float32),
        ],
    )(h, pie)


def _sc_body(dist2_hbm, ht_hbm, out_hbm, d_v, hist_v, bufa_v, bufb_v,
             sel_v, list_v, dst_v, sem_a, sem_b):
    cid = lax.axis_index("c")
    sid = lax.axis_index("s")
    wid = cid * NUM_SUBCORES + sid
    b = wid // TPB
    q = wid % TPB

    pltpu.sync_copy(dist2_hbm.at[b], d_v)

    iota = lax.iota(jnp.int32, L)
    ones = jnp.ones((L,), jnp.int32)
    zeros = jnp.zeros((L,), jnp.int32)

    def read_d(i):
        return d_v[pl.ds(i * L, L)]

    def level(read, cnt, shift, remaining, dst):
        for i in range(16):
            hist_v[pl.ds(i * L, L)] = zeros
        nch = (cnt + L - 1) // L

        def hbody(i, carry):
            v = read(i)
            act = (i * L + iota) < cnt
            digit = lax.shift_right_logical(v, shift) & 255
            plsc.addupdate_scatter(hist_v, [digit], ones, mask=act)
            return carry

        lax.fori_loop(0, nch, hbody, jnp.int32(0))

        def sbody(j, carry):
            base, beta, below = carry
            cm = base + plsc.cumsum(hist_v[pl.ds(j * L, L)])
            lt = cm < remaining
            beta = beta + jnp.max(plsc.all_reduce_population_count(lt))
            below = jnp.maximum(below, jnp.max(jnp.where(lt, cm, zeros)))
            return jnp.max(cm), beta, below

        _, beta, below = lax.fori_loop(
            0, 16, sbody, (jnp.int32(0), jnp.int32(0), jnp.int32(0)))
        remaining = remaining - below

        if dst is None:
            return beta, remaining, cnt

        def fbody(i, off):
            v = read(i)
            act = ((i * L + iota) < cnt) & (
                (lax.shift_right_logical(v, shift) & 255) == beta)
            plsc.store_compressed(dst.at[pl.ds(off, L)], v, mask=act)
            return off + jnp.max(plsc.all_reduce_population_count(act))

        newcnt = lax.fori_loop(0, nch, fbody, jnp.int32(0))
        return beta, remaining, newcnt

    read_a = lambda i: bufa_v[pl.ds(i * L, L)]
    read_b = lambda i: bufb_v[pl.ds(i * L, L)]

    beta0, rem, cnt1 = level(read_d, N, 24, jnp.int32(K), bufa_v)
    beta1, rem, cnt2 = level(read_a, cnt1, 16, rem, bufb_v)
    beta2, rem, cnt3 = level(read_b, cnt2, 8, rem, bufa_v)
    beta3, rem, _ = level(read_a, cnt3, 0, rem, None)
    vbits = ((beta0 * 256 + beta1) * 256 + beta2) * 256 + beta3

    def ebody(i, carry):
        offl, offt = carry
        v = read_d(i)
        lane = i * L + iota
        actl = v < vbits
        plsc.store_compressed(sel_v.at[pl.ds(offl, L)], lane, mask=actl)
        offl = offl + jnp.max(plsc.all_reduce_population_count(actl))
        actt = v == vbits
        plsc.store_compressed(bufb_v.at[pl.ds(offt, L)], lane, mask=actt)
        offt = offt + jnp.max(plsc.all_reduce_population_count(actt))
        return offl, offt

    n_less, _ = lax.fori_loop(0, NCH, ebody, (jnp.int32(0), jnp.int32(0)))

    def cbody(j, carry):
        sel_v[pl.ds(n_less + j * L, L)] = bufb_v[pl.ds(j * L, L)]
        return carry

    lax.fori_loop(0, (rem + L - 1) // L, cbody, jnp.int32(0))

    base2 = b * (2 * N)
    for s in range(2):
        for j in range(64 // L):
            kv = sel_v[pl.ds(q * CPT + s * 64 + j * L, L)]
            r = base2 + kv * 2
            pos = 2 * (j * L + iota)
            svec = jnp.broadcast_to(jnp.int32(s), (L,))
            plsc.store_scatter(list_v, [svec, pos], r)
            plsc.store_scatter(list_v, [svec, pos + 1], r + 1)
    pltpu.async_copy(ht_hbm.at[list_v.at[0]],
                     dst_v.at[pl.ds(0, 128)], sem_a)
    pltpu.async_copy(ht_hbm.at[list_v.at[1]],
                     dst_v.at[pl.ds(128, 128)], sem_b)
    pltpu.make_async_copy(ht_hbm.at[list_v.at[0]],
                          dst_v.at[pl.ds(0, 128)], sem_a).wait()
    pltpu.make_async_copy(ht_hbm.at[list_v.at[1]],
                          dst_v.at[pl.ds(128, 128)], sem_b).wait()
    pltpu.sync_copy(dst_v, out_hbm.at[b, pl.ds(q * 2 * CPT, 2 * CPT), :])


@functools.lru_cache(maxsize=1)
def _sc_topk_gather():
    mesh = plsc.VectorSubcoreMesh(
        core_axis_name="c", subcore_axis_name="s",
        num_cores=NUM_CORES, num_subcores=NUM_SUBCORES)
    return pl.kernel(
        _sc_body,
        out_type=jax.ShapeDtypeStruct((B, K * 2, 128), jnp.float32),
        mesh=mesh,
        compiler_params=pltpu.CompilerParams(
            needs_layout_passes=False, use_tc_tiling_on_sc=False),
        scratch_types=[
            pltpu.VMEM((N,), jnp.int32),
            pltpu.VMEM((256,), jnp.int32),
            pltpu.VMEM((N + L,), jnp.int32),
            pltpu.VMEM((N + L,), jnp.int32),
            pltpu.VMEM((K + 2 * L,), jnp.int32),
            pltpu.VMEM((2, 128), jnp.int32),
            pltpu.VMEM((2 * CPT, 128), jnp.float32),
            pltpu.SemaphoreType.DMA,
            pltpu.SemaphoreType.DMA,
        ],
    )


def _attn_body(nb_ref, pir_ref, wc_ref, bc_ref, wa_ref, ba_ref, out_ref):
    relT = nb_ref[0] - pir_ref[0]
    tT = lax.dot_general(relT, wc_ref[...], (((1,), (1,)), ((), ())),
                         preferred_element_type=jnp.float32)
    tT = tT + bc_ref[...][None, :]
    s = lax.dot_general(tT, tT, (((1,), (1,)), ((), ())),
                        preferred_element_type=jnp.float32)
    e = jnp.exp(s - jnp.max(s, axis=1, keepdims=True))
    z = jnp.sum(e, axis=1, keepdims=True)
    w = (jnp.sum(e / z, axis=0) * (1.0 / K))[None, :]
    feat = jnp.dot(w, tT, preferred_element_type=jnp.float32)
    o = lax.dot_general(feat, wa_ref[...], (((1,), (1,)), ((), ())),
                        preferred_element_type=jnp.float32)
    out_ref[0, 0, :] = jnp.maximum(o + ba_ref[...][None, :], 0.0)[0]


def _attn(nbrs, pir, w_conv, b_conv, w_att, b_att):
    return pl.pallas_call(
        _attn_body,
        grid=(B,),
        in_specs=[
            pl.BlockSpec((1, K, C), lambda b: (b, 0, 0)),
            pl.BlockSpec((1, 1, C), lambda b: (b, 0, 0)),
            pl.BlockSpec((C, C), lambda b: (0, 0)),
            pl.BlockSpec((C,), lambda b: (0,)),
            pl.BlockSpec((C, C), lambda b: (0, 0)),
            pl.BlockSpec((C,), lambda b: (0,)),
        ],
        out_specs=pl.BlockSpec((1, 1, C), lambda b: (b, 0, 0)),
        out_shape=jax.ShapeDtypeStruct((B, 1, C), jnp.float32),
    )(nbrs, pir, w_conv, b_conv, w_att, b_att)


def kernel(h, pi, W_conv, b_conv, W_att, b_att):
    pie = pi[:, :, None]
    dist2_r, ht = _dist2(h, pie)
    dist2 = dist2_r.reshape(B, N)
    ht2 = ht.reshape(B * N * 2, 128)
    nbrs = _sc_topk_gather()(dist2, ht2).reshape(B, K, C)
    pir = pi[:, None, :]
    return _attn(nbrs, pir, W_conv, b_conv, W_att, b_att).reshape(B, C)
